# baseline (speedup 1.0000x reference)
import functools

import jax
import jax.numpy as jnp
from jax import lax
from jax.experimental import pallas as pl
from jax.experimental.pallas import tpu as pltpu
from jax.experimental.pallas import tpu_sc as plsc

D = 128
NC = 2
NS = 16
NW = NC * NS
CH = 128
NBUF = 5


@functools.lru_cache(maxsize=None)
def _build(N, V):
    NPW = N // NW
    G = NPW // CH
    NGRP = G // NBUF
    assert G % NBUF == 0 and NGRP >= 2
    mesh = plsc.VectorSubcoreMesh(core_axis_name="c", subcore_axis_name="s")

    @functools.partial(
        pl.kernel,
        out_type=jax.ShapeDtypeStruct((N, D), jnp.float32),
        mesh=mesh,
        scratch_types=[
            pltpu.VMEM((G, CH), jnp.int32),
            pltpu.VMEM((NBUF, CH, D), jnp.float32),
        ] + [pltpu.SemaphoreType.DMA] * (2 * NBUF),
    )
    def emb_kernel(idx_hbm, emb_hbm, out_hbm, idx_v, bufs, *sems):
        gsems = sems[:NBUF]
        wsems = sems[NBUF:]
        wid = lax.axis_index("s") * NC + lax.axis_index("c")
        pltpu.sync_copy(idx_hbm.at[wid], idx_v)
        base = wid * NPW

        def g_copy(g, b):
            return pltpu.make_async_copy(
                emb_hbm.at[idx_v.at[g]], bufs.at[b], gsems[b])

        def w_start(g, b):
            pltpu.make_async_copy(
                bufs.at[b], out_hbm.at[pl.ds(base + g * CH, CH)],
                wsems[b]).start()

        def w_wait(b):
            pltpu.make_async_copy(
                out_hbm.at[pl.ds(0, CH)], bufs.at[b], wsems[b]).wait()

        def step(g, b, prefetch):
            pb = (b - 1) % NBUF
            w_wait(pb)
            if prefetch:
                g_copy(g - 1 + NBUF, pb).start()
            g_copy(g, b).wait()
            w_start(g, b)

        for b in range(NBUF):
            g_copy(b, b).start()
        g_copy(0, 0).wait()
        w_start(0, 0)

        def group(k, carry):
            g0 = 1 + k * NBUF
            for j in range(NBUF):
                step(g0 + j, (1 + j) % NBUF, prefetch=True)
            return carry

        lax.fori_loop(0, NGRP - 1, group, 0)

        for j in range(NBUF - 1):
            step(G - NBUF + 1 + j, (1 + j) % NBUF, prefetch=False)
        w_wait((NBUF - 1) % NBUF)

    return emb_kernel


def kernel(inputs, embedding):
    B, S = inputs.shape
    N = B * S
    V, d = embedding.shape
    idx = inputs.reshape(NW, (N // NW) // CH, CH).astype(jnp.int32)
    out = _build(N, V)(idx, embedding)
    return out.reshape(B, S, d)


# speedup vs baseline: 1.8572x; 1.8572x over previous
import functools

import jax
import jax.numpy as jnp
from jax import lax
from jax.experimental import pallas as pl
from jax.experimental.pallas import tpu as pltpu
from jax.experimental.pallas import tpu_sc as plsc

NC = 2
NS = 16
NW = NC * NS
IC = 128
NBUF = 8


@functools.lru_cache(maxsize=None)
def _build(B, S, V, D):
    RPW = B // NW
    NIG = RPW // IC
    NGRP = IC // NBUF
    assert RPW % IC == 0 and IC % NBUF == 0 and NGRP >= 2
    mesh = plsc.VectorSubcoreMesh(core_axis_name="c", subcore_axis_name="s")

    @functools.partial(
        pl.kernel,
        out_type=jax.ShapeDtypeStruct((B, S, D), jnp.float32),
        mesh=mesh,
        scratch_types=[
            pltpu.VMEM((2, IC, S), jnp.int32),
            pltpu.VMEM((NBUF, S, D), jnp.float32),
            pltpu.SemaphoreType.DMA,
            pltpu.SemaphoreType.DMA,
        ] + [pltpu.SemaphoreType.DMA] * (2 * NBUF),
        compiler_params=pltpu.CompilerParams(use_tc_tiling_on_sc=True),
    )
    def emb_kernel(idx_hbm, emb_hbm, out_hbm, idx_v, bufs, sA, sB, *sems):
        gsems = sems[:NBUF]
        wsems = sems[NBUF:]
        ssems = [sA, sB]
        wid = lax.axis_index("s") * NC + lax.axis_index("c")
        row0 = wid * RPW

        def stage(k):
            return pltpu.make_async_copy(
                idx_hbm.at[pl.ds(row0 + k * IC, IC)],
                idx_v.at[k % 2], ssems[k % 2])

        def g_copy(p, r, b):
            return pltpu.make_async_copy(
                emb_hbm.at[idx_v.at[p, r]], bufs.at[b], gsems[b])

        def w_start(row, b):
            pltpu.make_async_copy(
                bufs.at[b], out_hbm.at[row], wsems[b]).start()

        def w_wait(b):
            pltpu.make_async_copy(bufs.at[b], out_hbm.at[0], wsems[b]).wait()

        stage(0).start()
        for k in range(NIG):
            if k + 1 < NIG:
                stage(k + 1).start()
            stage(k).wait()
            p = k % 2
            rbase = row0 + k * IC

            def step(r, b, prefetch):
                pb = (b - 1) % NBUF
                w_wait(pb)
                if prefetch:
                    g_copy(p, r - 1 + NBUF, pb).start()
                g_copy(p, r, b).wait()
                w_start(rbase + r, b)

            for b in range(NBUF):
                g_copy(p, b, b).start()
            g_copy(p, 0, 0).wait()
            w_start(rbase, 0)

            def group(j, carry):
                r0 = 1 + j * NBUF
                for q in range(NBUF):
                    step(r0 + q, (1 + q) % NBUF, prefetch=True)
                return carry

            lax.fori_loop(0, NGRP - 1, group, 0)

            for q in range(NBUF - 1):
                step(IC - NBUF + 1 + q, (1 + q) % NBUF, prefetch=False)
            w_wait((NBUF - 1) % NBUF)

    return emb_kernel


def kernel(inputs, embedding):
    B, S = inputs.shape
    V, D = embedding.shape
    return _build(B, S, V, D)(inputs.astype(jnp.int32), embedding)
